# baseline (device time: 24627 ns/iter reference)
import jax
import jax.numpy as jnp
from jax import lax
from jax.experimental import pallas as pl
from jax.experimental.pallas import tpu as pltpu

KQ = 4


def kernel(x):
    m_per, n = x.shape
    q_rows = m_per // 4
    rows = q_rows // KQ

    def body(x_ref, out_ref, xs, xr, ys, yr, zs, zr):
        my_x = lax.axis_index("x")
        my_y = lax.axis_index("y")
        my_z = lax.axis_index("z")
        x_peer = (1 - my_x, my_y, my_z)
        y_peer = (my_x, my_y ^ 1, my_z)
        z_peer = (my_x, my_y, my_z ^ 1)

        p = 2 * (my_y % 2) + (my_z % 2)
        mine = my_x * m_per
        remote = (1 - my_x) * m_per

        def q_rows_at(base, q, k):
            return pl.ds(base + q * q_rows + k * rows, rows)

        barrier_sem = pltpu.get_barrier_semaphore()
        for peer in (x_peer, y_peer, z_peer):
            pl.semaphore_signal(
                barrier_sem, inc=1, device_id=peer,
                device_id_type=pl.DeviceIdType.MESH,
            )
        pl.semaphore_wait(barrier_sem, 3)

        out_ref[pl.ds(mine + p * q_rows, q_rows), :] = (
            x_ref[pl.ds(p * q_rows, q_rows), :].astype(jnp.bfloat16)
        )
        x_rdmas = []
        for k in range(KQ):
            r = pltpu.make_async_remote_copy(
                src_ref=out_ref.at[q_rows_at(mine, p, k)],
                dst_ref=out_ref.at[q_rows_at(mine, p, k)],
                send_sem=xs.at[k],
                recv_sem=xr.at[k],
                device_id=x_peer,
                device_id_type=pl.DeviceIdType.MESH,
            )
            r.start()
            x_rdmas.append(r)

        for dq in (1, 2, 3):
            q = p ^ dq
            out_ref[pl.ds(mine + q * q_rows, q_rows), :] = (
                x_ref[pl.ds(q * q_rows, q_rows), :].astype(jnp.bfloat16)
            )

        def fwd(slot_sems_s, slot_sems_r, slot, q, k, peer):
            r = pltpu.make_async_remote_copy(
                src_ref=out_ref.at[q_rows_at(remote, q, k)],
                dst_ref=out_ref.at[q_rows_at(remote, q, k)],
                send_sem=slot_sems_s.at[slot],
                recv_sem=slot_sems_r.at[slot],
                device_id=peer,
                device_id_type=pl.DeviceIdType.MESH,
            )
            r.start()
            return r

        def recv_wait(slot_sems_s, slot_sems_r, slot, q, k, peer):
            r = pltpu.make_async_remote_copy(
                src_ref=out_ref.at[q_rows_at(remote, q, k)],
                dst_ref=out_ref.at[q_rows_at(remote, q, k)],
                send_sem=slot_sems_s.at[slot],
                recv_sem=slot_sems_r.at[slot],
                device_id=peer,
                device_id_type=pl.DeviceIdType.MESH,
            )
            r.wait_recv()

        sends = list(x_rdmas)
        x_rdmas[0].wait_recv()
        sends.append(fwd(ys, yr, 0, p, 0, y_peer))
        sends.append(fwd(zs, zr, 0, p, 0, z_peer))
        x_rdmas[1].wait_recv()
        sends.append(fwd(ys, yr, 1, p, 1, y_peer))
        sends.append(fwd(zs, zr, 1, p, 1, z_peer))
        recv_wait(zs, zr, 0, p ^ 1, 0, z_peer)
        sends.append(fwd(ys, yr, 4, p ^ 1, 0, y_peer))
        x_rdmas[2].wait_recv()
        sends.append(fwd(ys, yr, 2, p, 2, y_peer))
        sends.append(fwd(zs, zr, 2, p, 2, z_peer))
        recv_wait(zs, zr, 1, p ^ 1, 1, z_peer)
        sends.append(fwd(ys, yr, 5, p ^ 1, 1, y_peer))
        x_rdmas[3].wait_recv()
        sends.append(fwd(ys, yr, 3, p, 3, y_peer))
        sends.append(fwd(zs, zr, 3, p, 3, z_peer))
        recv_wait(ys, yr, 2, p ^ 2, 2, y_peer)
        sends.append(fwd(zs, zr, 4, p ^ 2, 2, z_peer))
        recv_wait(ys, yr, 3, p ^ 2, 3, y_peer)
        sends.append(fwd(zs, zr, 5, p ^ 2, 3, z_peer))

        recv_wait(zs, zr, 2, p ^ 1, 2, z_peer)
        recv_wait(zs, zr, 3, p ^ 1, 3, z_peer)
        recv_wait(ys, yr, 0, p ^ 2, 0, y_peer)
        recv_wait(ys, yr, 1, p ^ 2, 1, y_peer)
        recv_wait(ys, yr, 4, p ^ 3, 0, y_peer)
        recv_wait(ys, yr, 5, p ^ 3, 1, y_peer)
        recv_wait(zs, zr, 4, p ^ 3, 2, z_peer)
        recv_wait(zs, zr, 5, p ^ 3, 3, z_peer)

        for r in sends:
            r.wait_send()

    return pl.pallas_call(
        body,
        out_shape=jax.ShapeDtypeStruct((2 * m_per, n), jnp.bfloat16),
        in_specs=[pl.BlockSpec(memory_space=pltpu.VMEM)],
        out_specs=pl.BlockSpec(memory_space=pltpu.VMEM),
        scratch_shapes=[
            pltpu.SemaphoreType.DMA((KQ,)),
            pltpu.SemaphoreType.DMA((KQ,)),
            pltpu.SemaphoreType.DMA((6,)),
            pltpu.SemaphoreType.DMA((6,)),
            pltpu.SemaphoreType.DMA((6,)),
            pltpu.SemaphoreType.DMA((6,)),
        ],
        compiler_params=pltpu.CompilerParams(collective_id=0),
    )(x)


# device time: 23509 ns/iter; 1.0476x vs baseline; 1.0476x over previous
import jax
import jax.numpy as jnp
from jax import lax
from jax.experimental import pallas as pl
from jax.experimental.pallas import tpu as pltpu

N_CHUNK = 8


def kernel(x):
    m_per, n = x.shape
    half_rows = m_per // 2
    rows = half_rows // N_CHUNK

    def body(x_hbm, out_hbm, x_vmem, mine_bf,
             load_sem, store_sem, x_send_sems, x_recv_sems,
             z_send_sems, z_recv_sems):
        my_x = lax.axis_index("x")
        my_y = lax.axis_index("y")
        my_z = lax.axis_index("z")
        x_peer = (1 - my_x, my_y, my_z)
        z_peer = (my_x, my_y, my_z ^ 1)

        half = my_z % 2
        mine = my_x * m_per
        remote = (1 - my_x) * m_per
        send_base = half * half_rows
        x_base = remote + half * half_rows
        z_base = remote + (1 - half) * half_rows

        load = pltpu.make_async_copy(x_hbm, x_vmem, load_sem)
        load.start()

        barrier_sem = pltpu.get_barrier_semaphore()
        for peer in (x_peer, z_peer):
            pl.semaphore_signal(
                barrier_sem, inc=1, device_id=peer,
                device_id_type=pl.DeviceIdType.MESH,
            )
        pl.semaphore_wait(barrier_sem, 2)
        load.wait()

        mine_bf[pl.ds(send_base, half_rows), :] = (
            x_vmem[pl.ds(send_base, half_rows), :].astype(jnp.bfloat16)
        )
        x_rdmas = []
        for k in range(N_CHUNK):
            r = pltpu.make_async_remote_copy(
                src_ref=mine_bf.at[pl.ds(send_base + k * rows, rows)],
                dst_ref=out_hbm.at[pl.ds(mine + send_base + k * rows, rows)],
                send_sem=x_send_sems.at[k],
                recv_sem=x_recv_sems.at[k],
                device_id=x_peer,
                device_id_type=pl.DeviceIdType.MESH,
            )
            r.start()
            x_rdmas.append(r)

        other = (1 - half) * half_rows
        mine_bf[pl.ds(other, half_rows), :] = (
            x_vmem[pl.ds(other, half_rows), :].astype(jnp.bfloat16)
        )
        store = pltpu.make_async_copy(
            mine_bf, out_hbm.at[pl.ds(mine, m_per)], store_sem
        )
        store.start()

        z_rdmas = []
        for k in range(N_CHUNK):
            x_rdmas[k].wait_recv()
            r = pltpu.make_async_remote_copy(
                src_ref=out_hbm.at[pl.ds(x_base + k * rows, rows)],
                dst_ref=out_hbm.at[pl.ds(x_base + k * rows, rows)],
                send_sem=z_send_sems.at[k],
                recv_sem=z_recv_sems.at[k],
                device_id=z_peer,
                device_id_type=pl.DeviceIdType.MESH,
            )
            r.start()
            z_rdmas.append(r)

        for k in range(N_CHUNK):
            recv_only = pltpu.make_async_remote_copy(
                src_ref=out_hbm.at[pl.ds(z_base + k * rows, rows)],
                dst_ref=out_hbm.at[pl.ds(z_base + k * rows, rows)],
                send_sem=z_send_sems.at[k],
                recv_sem=z_recv_sems.at[k],
                device_id=z_peer,
                device_id_type=pl.DeviceIdType.MESH,
            )
            recv_only.wait_recv()

        for k in range(N_CHUNK):
            x_rdmas[k].wait_send()
            z_rdmas[k].wait_send()
        store.wait()

    out_bf = jnp.bfloat16
    return pl.pallas_call(
        body,
        out_shape=jax.ShapeDtypeStruct((2 * m_per, n), out_bf),
        in_specs=[pl.BlockSpec(memory_space=pl.ANY)],
        out_specs=pl.BlockSpec(memory_space=pl.ANY),
        scratch_shapes=[
            pltpu.VMEM((m_per, n), jnp.float32),
            pltpu.VMEM((m_per, n), jnp.bfloat16),
            pltpu.SemaphoreType.DMA,
            pltpu.SemaphoreType.DMA,
            pltpu.SemaphoreType.DMA((N_CHUNK,)),
            pltpu.SemaphoreType.DMA((N_CHUNK,)),
            pltpu.SemaphoreType.DMA((N_CHUNK,)),
            pltpu.SemaphoreType.DMA((N_CHUNK,)),
        ],
        compiler_params=pltpu.CompilerParams(collective_id=0),
    )(x)


# device time: 23487 ns/iter; 1.0485x vs baseline; 1.0009x over previous
import jax
import jax.numpy as jnp
from jax import lax
from jax.experimental import pallas as pl
from jax.experimental.pallas import tpu as pltpu

N_CHUNK = 8


def kernel(x):
    m_per, n = x.shape
    half_rows = m_per // 2
    rows = half_rows // N_CHUNK

    def body(x_hbm, out_hbm, x_vmem, mine_bf,
             load_sem, load_sem2, store_sem, x_send_sems, x_recv_sems,
             z_send_sems, z_recv_sems):
        my_x = lax.axis_index("x")
        my_y = lax.axis_index("y")
        my_z = lax.axis_index("z")
        x_peer = (1 - my_x, my_y, my_z)
        z_peer = (my_x, my_y, my_z ^ 1)

        half = my_z % 2
        mine = my_x * m_per
        remote = (1 - my_x) * m_per
        send_base = half * half_rows
        x_base = remote + half * half_rows
        z_base = remote + (1 - half) * half_rows

        load_a = pltpu.make_async_copy(
            x_hbm.at[pl.ds(send_base, half_rows)],
            x_vmem.at[pl.ds(send_base, half_rows)],
            load_sem,
        )
        load_a.start()
        other = (1 - half) * half_rows
        load_b = pltpu.make_async_copy(
            x_hbm.at[pl.ds(other, half_rows)],
            x_vmem.at[pl.ds(other, half_rows)],
            load_sem2,
        )
        load_b.start()

        barrier_sem = pltpu.get_barrier_semaphore()
        for peer in (x_peer, z_peer):
            pl.semaphore_signal(
                barrier_sem, inc=1, device_id=peer,
                device_id_type=pl.DeviceIdType.MESH,
            )
        pl.semaphore_wait(barrier_sem, 2)
        load_a.wait()

        mine_bf[pl.ds(send_base, half_rows), :] = (
            x_vmem[pl.ds(send_base, half_rows), :].astype(jnp.bfloat16)
        )
        x_rdmas = []
        for k in range(N_CHUNK):
            r = pltpu.make_async_remote_copy(
                src_ref=mine_bf.at[pl.ds(send_base + k * rows, rows)],
                dst_ref=out_hbm.at[pl.ds(mine + send_base + k * rows, rows)],
                send_sem=x_send_sems.at[k],
                recv_sem=x_recv_sems.at[k],
                device_id=x_peer,
                device_id_type=pl.DeviceIdType.MESH,
            )
            r.start()
            x_rdmas.append(r)

        load_b.wait()
        mine_bf[pl.ds(other, half_rows), :] = (
            x_vmem[pl.ds(other, half_rows), :].astype(jnp.bfloat16)
        )
        store = pltpu.make_async_copy(
            mine_bf, out_hbm.at[pl.ds(mine, m_per)], store_sem
        )
        store.start()

        z_rdmas = []
        for k in range(N_CHUNK):
            x_rdmas[k].wait_recv()
            r = pltpu.make_async_remote_copy(
                src_ref=out_hbm.at[pl.ds(x_base + k * rows, rows)],
                dst_ref=out_hbm.at[pl.ds(x_base + k * rows, rows)],
                send_sem=z_send_sems.at[k],
                recv_sem=z_recv_sems.at[k],
                device_id=z_peer,
                device_id_type=pl.DeviceIdType.MESH,
            )
            r.start()
            z_rdmas.append(r)

        for k in range(N_CHUNK):
            recv_only = pltpu.make_async_remote_copy(
                src_ref=out_hbm.at[pl.ds(z_base + k * rows, rows)],
                dst_ref=out_hbm.at[pl.ds(z_base + k * rows, rows)],
                send_sem=z_send_sems.at[k],
                recv_sem=z_recv_sems.at[k],
                device_id=z_peer,
                device_id_type=pl.DeviceIdType.MESH,
            )
            recv_only.wait_recv()

        for k in range(N_CHUNK):
            x_rdmas[k].wait_send()
            z_rdmas[k].wait_send()
        store.wait()

    out_bf = jnp.bfloat16
    return pl.pallas_call(
        body,
        out_shape=jax.ShapeDtypeStruct((2 * m_per, n), out_bf),
        in_specs=[pl.BlockSpec(memory_space=pltpu.MemorySpace.HBM)],
        out_specs=pl.BlockSpec(memory_space=pltpu.MemorySpace.HBM),
        scratch_shapes=[
            pltpu.VMEM((m_per, n), jnp.float32),
            pltpu.VMEM((m_per, n), jnp.bfloat16),
            pltpu.SemaphoreType.DMA,
            pltpu.SemaphoreType.DMA,
            pltpu.SemaphoreType.DMA,
            pltpu.SemaphoreType.DMA((N_CHUNK,)),
            pltpu.SemaphoreType.DMA((N_CHUNK,)),
            pltpu.SemaphoreType.DMA((N_CHUNK,)),
            pltpu.SemaphoreType.DMA((N_CHUNK,)),
        ],
        compiler_params=pltpu.CompilerParams(collective_id=0),
    )(x)


# device time: 22767 ns/iter; 1.0817x vs baseline; 1.0316x over previous
import jax
import jax.numpy as jnp
from jax import lax
from jax.experimental import pallas as pl
from jax.experimental.pallas import tpu as pltpu

N_CHUNK = 16


def kernel(x):
    m_per, n = x.shape
    half_rows = m_per // 2
    rows = half_rows // N_CHUNK

    def body(x_ref, out_ref, x_send_sems, x_recv_sems, z_send_sems,
             z_recv_sems):
        my_x = lax.axis_index("x")
        my_y = lax.axis_index("y")
        my_z = lax.axis_index("z")
        x_peer = (1 - my_x, my_y, my_z)
        z_peer = (my_x, my_y, my_z ^ 1)

        half = my_z % 2
        mine = my_x * m_per
        send_base = mine + half * half_rows
        x_base = (1 - my_x) * m_per + half * half_rows
        z_base = (1 - my_x) * m_per + (1 - half) * half_rows

        barrier_sem = pltpu.get_barrier_semaphore()
        for peer in (x_peer, z_peer):
            pl.semaphore_signal(
                barrier_sem, inc=1, device_id=peer,
                device_id_type=pl.DeviceIdType.MESH,
            )
        pl.semaphore_wait(barrier_sem, 2)

        x_rdmas = []
        for piece in range(2):
            p_off = piece * (half_rows // 2)
            out_ref[pl.ds(send_base + p_off, half_rows // 2), :] = (
                x_ref[pl.ds(half * half_rows + p_off, half_rows // 2), :]
                .astype(jnp.bfloat16)
            )
            for k in range(piece * N_CHUNK // 2, (piece + 1) * N_CHUNK // 2):
                r = pltpu.make_async_remote_copy(
                    src_ref=out_ref.at[pl.ds(send_base + k * rows, rows)],
                    dst_ref=out_ref.at[pl.ds(send_base + k * rows, rows)],
                    send_sem=x_send_sems.at[k],
                    recv_sem=x_recv_sems.at[k],
                    device_id=x_peer,
                    device_id_type=pl.DeviceIdType.MESH,
                )
                r.start()
                x_rdmas.append(r)

        other = 1 - half
        out_ref[pl.ds(mine + other * half_rows, half_rows), :] = (
            x_ref[pl.ds(other * half_rows, half_rows), :].astype(jnp.bfloat16)
        )

        z_rdmas = []
        for k in range(N_CHUNK):
            x_rdmas[k].wait_recv()
            r = pltpu.make_async_remote_copy(
                src_ref=out_ref.at[pl.ds(x_base + k * rows, rows)],
                dst_ref=out_ref.at[pl.ds(x_base + k * rows, rows)],
                send_sem=z_send_sems.at[k],
                recv_sem=z_recv_sems.at[k],
                device_id=z_peer,
                device_id_type=pl.DeviceIdType.MESH,
            )
            r.start()
            z_rdmas.append(r)

        for k in range(N_CHUNK):
            recv_only = pltpu.make_async_remote_copy(
                src_ref=out_ref.at[pl.ds(z_base + k * rows, rows)],
                dst_ref=out_ref.at[pl.ds(z_base + k * rows, rows)],
                send_sem=z_send_sems.at[k],
                recv_sem=z_recv_sems.at[k],
                device_id=z_peer,
                device_id_type=pl.DeviceIdType.MESH,
            )
            recv_only.wait_recv()

        for k in range(N_CHUNK):
            x_rdmas[k].wait_send()
            z_rdmas[k].wait_send()

    return pl.pallas_call(
        body,
        out_shape=jax.ShapeDtypeStruct((2 * m_per, n), jnp.bfloat16),
        in_specs=[pl.BlockSpec(memory_space=pltpu.VMEM)],
        out_specs=pl.BlockSpec(memory_space=pltpu.VMEM),
        scratch_shapes=[
            pltpu.SemaphoreType.DMA((N_CHUNK,)),
            pltpu.SemaphoreType.DMA((N_CHUNK,)),
            pltpu.SemaphoreType.DMA((N_CHUNK,)),
            pltpu.SemaphoreType.DMA((N_CHUNK,)),
        ],
        compiler_params=pltpu.CompilerParams(collective_id=0),
    )(x)
